# baseline (speedup 1.0000x reference)
import functools

import jax
import jax.numpy as jnp
from jax import lax
from jax.experimental import pallas as pl
from jax.experimental.pallas import tpu as pltpu
from jax.experimental.pallas import tpu_sc as plsc

T = 4096
D = 1024
ED = 128
E = 8
H = 1024
TBLK = 512
NTB = T // TBLK
BLK = 256
NBLK = 23
CAP = NBLK * BLK
NW = 32
TPW = T // NW
CH = 32
NCH = TPW // CH
_HI = jax.lax.Precision.HIGHEST


def _router_body(e_ref, x_ref, w_ref,
                 pos_ref, gate_ref, be_ref, l1_ref, imp_ref,
                 P_s, O_s, R_s, c_s):
    i = pl.program_id(0)
    rin = jnp.concatenate([e_ref[...], x_ref[...]], axis=1)
    logits = jax.lax.dot(rin, w_ref[...])
    m = jnp.max(logits, axis=1, keepdims=True)
    ez = jnp.exp(logits - m)
    p = ez / jnp.sum(ez, axis=1, keepdims=True)
    P_s[pl.ds(i * TBLK, TBLK), :] = p
    pm = jnp.max(p, axis=1, keepdims=True)
    lane = lax.broadcasted_iota(jnp.int32, (TBLK, E), 1)
    first = jnp.min(jnp.where(p >= pm, lane, E), axis=1, keepdims=True)
    O = (lane == first).astype(jnp.float32)
    O_s[pl.ds(i * TBLK, TBLK), :] = O

    @pl.when(i == 0)
    def _():
        c_s[...] = jnp.zeros_like(c_s)

    r_i = lax.broadcasted_iota(jnp.int32, (TBLK, TBLK), 0)
    c_i = lax.broadcasted_iota(jnp.int32, (TBLK, TBLK), 1)
    tri = (c_i < r_i).astype(jnp.float32)
    prefix = jax.lax.dot(tri, O)
    R_s[pl.ds(i * TBLK, TBLK), :] = c_s[0:1, :] + prefix
    c_s[0:1, :] = c_s[0:1, :] + jnp.sum(O, axis=0, keepdims=True)

    @pl.when(i == NTB - 1)
    def _():
        P = P_s[...]
        Of = O_s[...]
        counts = c_s[0:1, :]
        padded = jnp.ceil(counts / BLK) * BLK
        ut = (lax.broadcasted_iota(jnp.int32, (E, E), 0)
              < lax.broadcasted_iota(jnp.int32, (E, E), 1)).astype(jnp.float32)
        off = jax.lax.dot(padded, ut, precision=_HI)
        pos_col = jnp.sum((R_s[...] + off) * Of, axis=1, keepdims=True)
        gate_col = jnp.max(P, axis=1, keepdims=True)
        t_r = lax.broadcasted_iota(jnp.int32, (NW, T), 1)
        i_r = lax.broadcasted_iota(jnp.int32, (NW, T), 0)
        m1 = (t_r // TPW == i_r).astype(jnp.float32)
        t_c = lax.broadcasted_iota(jnp.int32, (T, TPW), 0)
        l_c = lax.broadcasted_iota(jnp.int32, (T, TPW), 1)
        m2 = (t_c % TPW == l_c).astype(jnp.float32)
        pos2d = jax.lax.dot(m1, pos_col * m2, precision=_HI)
        pos_ref[...] = (pos2d + 0.5).astype(jnp.int32)
        gate_ref[...] = jnp.broadcast_to(gate_col, (T, 128))
        i8 = (lax.broadcasted_iota(jnp.int32, (E, E), 0)
              == lax.broadcasted_iota(jnp.int32, (E, E), 1)).astype(jnp.float32)
        off_col = jax.lax.dot_general(i8, off, (((1,), (1,)), ((), ())),
                                      precision=_HI)
        offb = jnp.broadcast_to(off_col, (E, 128))
        jblk = lax.broadcasted_iota(jnp.int32, (E, 128), 1).astype(jnp.float32) * BLK
        ind = (offb <= jblk).astype(jnp.float32)
        be = jnp.sum(ind, axis=0, keepdims=True) - 1.0
        nused = jnp.sum(padded) / BLK
        lane128 = lax.broadcasted_iota(jnp.int32, (1, 128), 1)
        be = jnp.where(lane128 == NBLK, nused, be)
        be_ref[...] = jnp.broadcast_to(be, (E, 128)).astype(jnp.int32)
        l1_ref[...] = jnp.reshape(jnp.sum(P) / T, (1, 1))
        imp = jnp.sum(P, axis=0, keepdims=True)
        mu = jnp.sum(imp) / E
        var = jnp.sum((imp - mu) ** 2) / E
        imp_ref[...] = jnp.reshape(var / (mu * mu + 1e-10), (1, 1))


_router_call = pl.pallas_call(
    _router_body,
    grid=(NTB,),
    in_specs=[
        pl.BlockSpec((TBLK, ED), lambda i: (i, 0)),
        pl.BlockSpec((TBLK, D), lambda i: (i, 0)),
        pl.BlockSpec((ED + D, E), lambda i: (0, 0)),
    ],
    out_specs=[
        pl.BlockSpec((NW, TPW), lambda i: (0, 0)),
        pl.BlockSpec((T, 128), lambda i: (0, 0)),
        pl.BlockSpec((E, 128), lambda i: (0, 0)),
        pl.BlockSpec((1, 1), lambda i: (0, 0)),
        pl.BlockSpec((1, 1), lambda i: (0, 0)),
    ],
    out_shape=[
        jax.ShapeDtypeStruct((NW, TPW), jnp.int32),
        jax.ShapeDtypeStruct((T, 128), jnp.float32),
        jax.ShapeDtypeStruct((E, 128), jnp.int32),
        jax.ShapeDtypeStruct((1, 1), jnp.float32),
        jax.ShapeDtypeStruct((1, 1), jnp.float32),
    ],
    scratch_shapes=[
        pltpu.VMEM((T, E), jnp.float32),
        pltpu.VMEM((T, E), jnp.float32),
        pltpu.VMEM((T, E), jnp.float32),
        pltpu.VMEM((8, E), jnp.float32),
    ],
)


@functools.cache
def _sc_kernels():
    mesh = plsc.VectorSubcoreMesh(core_axis_name="c", subcore_axis_name="s")

    @functools.partial(
        pl.kernel,
        out_type=(jax.ShapeDtypeStruct((CAP, D), jnp.float32),
                  jax.ShapeDtypeStruct((CAP, 128), jnp.float32)),
        mesh=mesh,
        scratch_types=[
            pltpu.VMEM((NCH, CH), jnp.int32),
            pltpu.VMEM((TPW,), jnp.int32),
            pltpu.VMEM((CH, D), jnp.float32),
            pltpu.VMEM((CH, D), jnp.float32),
            pltpu.VMEM((TPW, 128), jnp.float32),
            pltpu.SemaphoreType.DMA,
            pltpu.SemaphoreType.DMA,
            pltpu.SemaphoreType.DMA,
            pltpu.SemaphoreType.DMA,
            pltpu.SemaphoreType.DMA,
        ],
    )
    def _sc_dispatch(pos_hbm, posf_hbm, x_hbm, gate_hbm, xs_hbm, gs_hbm,
                     pos_v, posf_v, xv0, xv1, gv,
                     lsem0, lsem1, ssem0, ssem1, gsem):
        wid = lax.axis_index("s") * 2 + lax.axis_index("c")
        base = wid * TPW
        pltpu.sync_copy(pos_hbm.at[wid], pos_v)
        pltpu.sync_copy(posf_hbm.at[pl.ds(base, TPW)], posf_v)
        pltpu.sync_copy(gate_hbm.at[pl.ds(base, TPW)], gv)
        gate_st = pltpu.async_copy(gv, gs_hbm.at[posf_v], gsem)
        bufs = (xv0, xv1)
        lsems = (lsem0, lsem1)
        ssems = (ssem0, ssem1)
        loads = {0: pltpu.async_copy(x_hbm.at[pl.ds(base, CH)], xv0, lsem0)}
        stores = {}
        for c in range(NCH):
            b = c % 2
            if c + 1 < NCH:
                if c - 1 >= 0:
                    stores[c - 1].wait()
                loads[c + 1] = pltpu.async_copy(
                    x_hbm.at[pl.ds(base + (c + 1) * CH, CH)],
                    bufs[1 - b], lsems[1 - b])
            loads[c].wait()
            stores[c] = pltpu.async_copy(bufs[b], xs_hbm.at[pos_v.at[c]], ssems[b])
        stores[NCH - 2].wait()
        stores[NCH - 1].wait()
        gate_st.wait()

    @functools.partial(
        pl.kernel,
        out_type=jax.ShapeDtypeStruct((T, D), jnp.float32),
        mesh=mesh,
        scratch_types=[
            pltpu.VMEM((NCH, CH), jnp.int32),
            pltpu.VMEM((CH, D), jnp.float32),
            pltpu.VMEM((CH, D), jnp.float32),
            pltpu.SemaphoreType.DMA,
            pltpu.SemaphoreType.DMA,
            pltpu.SemaphoreType.DMA,
            pltpu.SemaphoreType.DMA,
        ],
    )
    def _sc_combine(pos_hbm, ys_hbm, out_hbm, pos_v, yv0, yv1,
                    lsem0, lsem1, ssem0, ssem1):
        wid = lax.axis_index("s") * 2 + lax.axis_index("c")
        base = wid * TPW
        pltpu.sync_copy(pos_hbm.at[wid], pos_v)
        bufs = (yv0, yv1)
        lsems = (lsem0, lsem1)
        ssems = (ssem0, ssem1)
        loads = {0: pltpu.async_copy(ys_hbm.at[pos_v.at[0]], yv0, lsem0)}
        stores = {}
        for c in range(NCH):
            b = c % 2
            if c + 1 < NCH:
                if c - 1 >= 0:
                    stores[c - 1].wait()
                loads[c + 1] = pltpu.async_copy(
                    ys_hbm.at[pos_v.at[c + 1]], bufs[1 - b], lsems[1 - b])
            loads[c].wait()
            stores[c] = pltpu.async_copy(
                bufs[b], out_hbm.at[pl.ds(base + c * CH, CH)], ssems[b])
        stores[NCH - 2].wait()
        stores[NCH - 1].wait()

    return _sc_dispatch, _sc_combine


def _ffn_body(be_ref, xs_ref, gs_ref, w1_ref, b1_ref, w2_ref, b2_ref, ys_ref):
    i = pl.program_id(0)

    @pl.when(i < be_ref[NBLK])
    def _():
        xb = xs_ref[...]
        h = jnp.maximum(jax.lax.dot(xb, w1_ref[0]) + b1_ref[0], 0.0)
        y = jax.lax.dot(h, w2_ref[0]) + b2_ref[0]
        ys_ref[...] = y * gs_ref[:, 0:1]


def _data_idx(i, be):
    return jnp.where(i < be[NBLK], i, 0)


def _w_idx(i, be):
    return be[jnp.where(i < be[NBLK], i, be[NBLK] - 1)]


_ffn_call = pl.pallas_call(
    _ffn_body,
    grid_spec=pltpu.PrefetchScalarGridSpec(
        num_scalar_prefetch=1,
        grid=(NBLK,),
        in_specs=[
            pl.BlockSpec((BLK, D), lambda i, be: (_data_idx(i, be), 0)),
            pl.BlockSpec((BLK, 128), lambda i, be: (_data_idx(i, be), 0)),
            pl.BlockSpec((1, D, H), lambda i, be: (_w_idx(i, be), 0, 0)),
            pl.BlockSpec((1, 1, H), lambda i, be: (_w_idx(i, be), 0, 0)),
            pl.BlockSpec((1, H, D), lambda i, be: (_w_idx(i, be), 0, 0)),
            pl.BlockSpec((1, 1, D), lambda i, be: (_w_idx(i, be), 0, 0)),
        ],
        out_specs=pl.BlockSpec((BLK, D), lambda i, be: (i, 0)),
    ),
    out_shape=jax.ShapeDtypeStruct((CAP, D), jnp.float32),
)


def kernel(inputs, embed, router_weights, w1, b1, w2, b2):
    b, s, d = inputs.shape
    x = inputs.reshape(T, D)
    e2 = embed.reshape(T, ED)
    pos2d, gate_w, be2d, l1, imp = _router_call(e2, x, router_weights)
    pos3 = pos2d.reshape(NW, NCH, CH)
    posf = pos2d.reshape(T)
    be = be2d[0, :NBLK + 1]
    _sc_dispatch, _sc_combine = _sc_kernels()
    xs, gs = _sc_dispatch(pos3, posf, x, gate_w)
    ys = _ffn_call(be, xs, gs, w1, b1.reshape(E, 1, H), w2, b2.reshape(E, 1, D))
    out = _sc_combine(pos3, ys)
    return (out.reshape(b, s, d), l1[0, 0], imp[0, 0])


# speedup vs baseline: 1.0344x; 1.0344x over previous
import functools

import jax
import jax.numpy as jnp
from jax import lax
from jax.experimental import pallas as pl
from jax.experimental.pallas import tpu as pltpu
from jax.experimental.pallas import tpu_sc as plsc

T = 4096
D = 1024
ED = 128
E = 8
H = 1024
TBLK = 512
NTB = T // TBLK
BLK = 512
NBLK = 15
CAP = NBLK * BLK
NW = 32
TPW = T // NW
CH = 32
NCH = TPW // CH
_HI = jax.lax.Precision.HIGHEST


def _router_body(e_ref, x_ref, w_ref,
                 pos_ref, gate_ref, be_ref, l1_ref, imp_ref,
                 P_s, O_s, R_s, c_s):
    i = pl.program_id(0)
    rin = jnp.concatenate([e_ref[...], x_ref[...]], axis=1)
    logits = jax.lax.dot(rin, w_ref[...])
    m = jnp.max(logits, axis=1, keepdims=True)
    ez = jnp.exp(logits - m)
    p = ez / jnp.sum(ez, axis=1, keepdims=True)
    P_s[pl.ds(i * TBLK, TBLK), :] = p
    pm = jnp.max(p, axis=1, keepdims=True)
    lane = lax.broadcasted_iota(jnp.int32, (TBLK, E), 1)
    first = jnp.min(jnp.where(p >= pm, lane, E), axis=1, keepdims=True)
    O = (lane == first).astype(jnp.float32)
    O_s[pl.ds(i * TBLK, TBLK), :] = O

    @pl.when(i == 0)
    def _():
        c_s[...] = jnp.zeros_like(c_s)

    r_i = lax.broadcasted_iota(jnp.int32, (TBLK, TBLK), 0)
    c_i = lax.broadcasted_iota(jnp.int32, (TBLK, TBLK), 1)
    tri = (c_i < r_i).astype(jnp.float32)
    prefix = jax.lax.dot(tri, O)
    R_s[pl.ds(i * TBLK, TBLK), :] = c_s[0:1, :] + prefix
    c_s[0:1, :] = c_s[0:1, :] + jnp.sum(O, axis=0, keepdims=True)

    @pl.when(i == NTB - 1)
    def _():
        P = P_s[...]
        Of = O_s[...]
        counts = c_s[0:1, :]
        padded = jnp.ceil(counts / BLK) * BLK
        ut = (lax.broadcasted_iota(jnp.int32, (E, E), 0)
              < lax.broadcasted_iota(jnp.int32, (E, E), 1)).astype(jnp.float32)
        off = jax.lax.dot(padded, ut, precision=_HI)
        pos_col = jnp.sum((R_s[...] + off) * Of, axis=1, keepdims=True)
        gate_col = jnp.max(P, axis=1, keepdims=True)
        t_r = lax.broadcasted_iota(jnp.int32, (NW, T), 1)
        i_r = lax.broadcasted_iota(jnp.int32, (NW, T), 0)
        m1 = (t_r // TPW == i_r).astype(jnp.float32)
        t_c = lax.broadcasted_iota(jnp.int32, (T, TPW), 0)
        l_c = lax.broadcasted_iota(jnp.int32, (T, TPW), 1)
        m2 = (t_c % TPW == l_c).astype(jnp.float32)
        pos2d = jax.lax.dot(m1, pos_col * m2, precision=_HI)
        pos_ref[...] = (pos2d + 0.5).astype(jnp.int32)
        gate_ref[...] = jnp.broadcast_to(gate_col, (T, 128))
        i8 = (lax.broadcasted_iota(jnp.int32, (E, E), 0)
              == lax.broadcasted_iota(jnp.int32, (E, E), 1)).astype(jnp.float32)
        off_col = jax.lax.dot_general(i8, off, (((1,), (1,)), ((), ())),
                                      precision=_HI)
        offb = jnp.broadcast_to(off_col, (E, 128))
        jblk = lax.broadcasted_iota(jnp.int32, (E, 128), 1).astype(jnp.float32) * BLK
        ind = (offb <= jblk).astype(jnp.float32)
        be = jnp.sum(ind, axis=0, keepdims=True) - 1.0
        nused = jnp.sum(padded) / BLK
        lane128 = lax.broadcasted_iota(jnp.int32, (1, 128), 1)
        be = jnp.where(lane128 == NBLK, nused, be)
        be_ref[...] = jnp.broadcast_to(be, (E, 128)).astype(jnp.int32)
        l1_ref[...] = jnp.reshape(jnp.sum(P) / T, (1, 1))
        imp = jnp.sum(P, axis=0, keepdims=True)
        mu = jnp.sum(imp) / E
        var = jnp.sum((imp - mu) ** 2) / E
        imp_ref[...] = jnp.reshape(var / (mu * mu + 1e-10), (1, 1))


_router_call = pl.pallas_call(
    _router_body,
    grid=(NTB,),
    in_specs=[
        pl.BlockSpec((TBLK, ED), lambda i: (i, 0)),
        pl.BlockSpec((TBLK, D), lambda i: (i, 0)),
        pl.BlockSpec((ED + D, E), lambda i: (0, 0)),
    ],
    out_specs=[
        pl.BlockSpec((NW, TPW), lambda i: (0, 0)),
        pl.BlockSpec((T, 128), lambda i: (0, 0)),
        pl.BlockSpec((E, 128), lambda i: (0, 0)),
        pl.BlockSpec((1, 1), lambda i: (0, 0)),
        pl.BlockSpec((1, 1), lambda i: (0, 0)),
    ],
    out_shape=[
        jax.ShapeDtypeStruct((NW, TPW), jnp.int32),
        jax.ShapeDtypeStruct((T, 128), jnp.float32),
        jax.ShapeDtypeStruct((E, 128), jnp.int32),
        jax.ShapeDtypeStruct((1, 1), jnp.float32),
        jax.ShapeDtypeStruct((1, 1), jnp.float32),
    ],
    scratch_shapes=[
        pltpu.VMEM((T, E), jnp.float32),
        pltpu.VMEM((T, E), jnp.float32),
        pltpu.VMEM((T, E), jnp.float32),
        pltpu.VMEM((8, E), jnp.float32),
    ],
)


@functools.cache
def _sc_kernels():
    mesh = plsc.VectorSubcoreMesh(core_axis_name="c", subcore_axis_name="s")

    @functools.partial(
        pl.kernel,
        out_type=(jax.ShapeDtypeStruct((CAP, D), jnp.float32),
                  jax.ShapeDtypeStruct((CAP, 128), jnp.float32)),
        mesh=mesh,
        scratch_types=[
            pltpu.VMEM((NCH, CH), jnp.int32),
            pltpu.VMEM((TPW,), jnp.int32),
            pltpu.VMEM((CH, D), jnp.float32),
            pltpu.VMEM((CH, D), jnp.float32),
            pltpu.VMEM((TPW, 128), jnp.float32),
            pltpu.SemaphoreType.DMA,
            pltpu.SemaphoreType.DMA,
            pltpu.SemaphoreType.DMA,
            pltpu.SemaphoreType.DMA,
            pltpu.SemaphoreType.DMA,
        ],
    )
    def _sc_dispatch(pos_hbm, posf_hbm, x_hbm, gate_hbm, xs_hbm, gs_hbm,
                     pos_v, posf_v, xv0, xv1, gv,
                     lsem0, lsem1, ssem0, ssem1, gsem):
        wid = lax.axis_index("s") * 2 + lax.axis_index("c")
        base = wid * TPW
        pltpu.sync_copy(pos_hbm.at[wid], pos_v)
        pltpu.sync_copy(posf_hbm.at[pl.ds(base, TPW)], posf_v)
        pltpu.sync_copy(gate_hbm.at[pl.ds(base, TPW)], gv)
        gate_st = pltpu.async_copy(gv, gs_hbm.at[posf_v], gsem)
        bufs = (xv0, xv1)
        lsems = (lsem0, lsem1)
        ssems = (ssem0, ssem1)
        loads = {0: pltpu.async_copy(x_hbm.at[pl.ds(base, CH)], xv0, lsem0)}
        stores = {}
        for c in range(NCH):
            b = c % 2
            if c + 1 < NCH:
                if c - 1 >= 0:
                    stores[c - 1].wait()
                loads[c + 1] = pltpu.async_copy(
                    x_hbm.at[pl.ds(base + (c + 1) * CH, CH)],
                    bufs[1 - b], lsems[1 - b])
            loads[c].wait()
            stores[c] = pltpu.async_copy(bufs[b], xs_hbm.at[pos_v.at[c]], ssems[b])
        stores[NCH - 2].wait()
        stores[NCH - 1].wait()
        gate_st.wait()

    @functools.partial(
        pl.kernel,
        out_type=jax.ShapeDtypeStruct((T, D), jnp.float32),
        mesh=mesh,
        scratch_types=[
            pltpu.VMEM((NCH, CH), jnp.int32),
            pltpu.VMEM((CH, D), jnp.float32),
            pltpu.VMEM((CH, D), jnp.float32),
            pltpu.SemaphoreType.DMA,
            pltpu.SemaphoreType.DMA,
            pltpu.SemaphoreType.DMA,
            pltpu.SemaphoreType.DMA,
        ],
    )
    def _sc_combine(pos_hbm, ys_hbm, out_hbm, pos_v, yv0, yv1,
                    lsem0, lsem1, ssem0, ssem1):
        wid = lax.axis_index("s") * 2 + lax.axis_index("c")
        base = wid * TPW
        pltpu.sync_copy(pos_hbm.at[wid], pos_v)
        bufs = (yv0, yv1)
        lsems = (lsem0, lsem1)
        ssems = (ssem0, ssem1)
        loads = {0: pltpu.async_copy(ys_hbm.at[pos_v.at[0]], yv0, lsem0)}
        stores = {}
        for c in range(NCH):
            b = c % 2
            if c + 1 < NCH:
                if c - 1 >= 0:
                    stores[c - 1].wait()
                loads[c + 1] = pltpu.async_copy(
                    ys_hbm.at[pos_v.at[c + 1]], bufs[1 - b], lsems[1 - b])
            loads[c].wait()
            stores[c] = pltpu.async_copy(
                bufs[b], out_hbm.at[pl.ds(base + c * CH, CH)], ssems[b])
        stores[NCH - 2].wait()
        stores[NCH - 1].wait()

    return _sc_dispatch, _sc_combine


def _ffn_body(be_ref, xs_ref, gs_ref, w1_ref, b1_ref, w2_ref, b2_ref, ys_ref):
    i = pl.program_id(0)

    @pl.when(i < be_ref[NBLK])
    def _():
        xb = xs_ref[...]
        h = jnp.maximum(jax.lax.dot(xb, w1_ref[0]) + b1_ref[0], 0.0)
        y = jax.lax.dot(h, w2_ref[0]) + b2_ref[0]
        ys_ref[...] = y * gs_ref[:, 0:1]


def _data_idx(i, be):
    return jnp.where(i < be[NBLK], i, 0)


def _w_idx(i, be):
    return be[jnp.where(i < be[NBLK], i, be[NBLK] - 1)]


_ffn_call = pl.pallas_call(
    _ffn_body,
    grid_spec=pltpu.PrefetchScalarGridSpec(
        num_scalar_prefetch=1,
        grid=(NBLK,),
        in_specs=[
            pl.BlockSpec((BLK, D), lambda i, be: (_data_idx(i, be), 0)),
            pl.BlockSpec((BLK, 128), lambda i, be: (_data_idx(i, be), 0)),
            pl.BlockSpec((1, D, H), lambda i, be: (_w_idx(i, be), 0, 0)),
            pl.BlockSpec((1, 1, H), lambda i, be: (_w_idx(i, be), 0, 0)),
            pl.BlockSpec((1, H, D), lambda i, be: (_w_idx(i, be), 0, 0)),
            pl.BlockSpec((1, 1, D), lambda i, be: (_w_idx(i, be), 0, 0)),
        ],
        out_specs=pl.BlockSpec((BLK, D), lambda i, be: (i, 0)),
    ),
    out_shape=jax.ShapeDtypeStruct((CAP, D), jnp.float32),
)


def kernel(inputs, embed, router_weights, w1, b1, w2, b2):
    b, s, d = inputs.shape
    x = inputs.reshape(T, D)
    e2 = embed.reshape(T, ED)
    pos2d, gate_w, be2d, l1, imp = _router_call(e2, x, router_weights)
    pos3 = pos2d.reshape(NW, NCH, CH)
    posf = pos2d.reshape(T)
    be = be2d[0, :NBLK + 1]
    _sc_dispatch, _sc_combine = _sc_kernels()
    xs, gs = _sc_dispatch(pos3, posf, x, gate_w)
    ys = _ffn_call(be, xs, gs, w1, b1.reshape(E, 1, H), w2, b2.reshape(E, 1, D))
    out = _sc_combine(pos3, ys)
    return (out.reshape(b, s, d), l1[0, 0], imp[0, 0])
